# Initial kernel scaffold; baseline (speedup 1.0000x reference)
import jax
import jax.numpy as jnp
from jax.experimental import pallas as pl


def kernel(t, x_mask, noising_mask, rigids_7, residx, edge_index, seq_edge_index, params):
    raise NotImplementedError("write your pallas kernel here")


# speedup vs baseline: 1.0758x; 1.0758x over previous
import functools

import jax
import jax.numpy as jnp
import numpy as np
from jax.experimental import pallas as pl

C_S = 256; C_Z = 64; C_HID = 16; H = 8; QK = 4; V = 8; NL = 2; HT = 64; SH = 128


def _ceil_to(x, m):
    return (x + m - 1) // m * m


def _pad_rows(x, mp):
    m = x.shape[0]
    if m == mp:
        return x
    return jnp.pad(x, ((0, mp - m),) + ((0, 0),) * (x.ndim - 1))


def _mm_kernel(x_ref, w_ref, b_ref, o_ref, *, act):
    acc = jnp.dot(x_ref[...], w_ref[...], preferred_element_type=jnp.float32)
    acc = acc + b_ref[...]
    if act == "relu":
        acc = jnp.maximum(acc, 0.0)
    o_ref[...] = acc


def pmm(x, w, b=None, act="none", bm=1024):
    m, k = x.shape
    n = w.shape[1]
    mp = _ceil_to(m, bm)
    xp = _pad_rows(x, mp)
    if b is None:
        b = jnp.zeros((n,), jnp.float32)
    b2 = b.reshape(1, n)
    out = pl.pallas_call(
        functools.partial(_mm_kernel, act=act),
        grid=(mp // bm,),
        in_specs=[
            pl.BlockSpec((bm, k), lambda i: (i, 0)),
            pl.BlockSpec((k, n), lambda i: (0, 0)),
            pl.BlockSpec((1, n), lambda i: (0, 0)),
        ],
        out_specs=pl.BlockSpec((bm, n), lambda i: (i, 0)),
        out_shape=jax.ShapeDtypeStruct((mp, n), jnp.float32),
    )(xp, w, b2)
    return out[:m]


def _node_trans_kernel(x_ref, w1_ref, b1_ref, w2_ref, b2_ref, w3_ref, b3_ref,
                       g_ref, be_ref, o_ref):
    x = x_ref[...]
    h = jnp.maximum(jnp.dot(x, w1_ref[...], preferred_element_type=jnp.float32)
                    + b1_ref[...], 0.0)
    h = jnp.maximum(jnp.dot(h, w2_ref[...], preferred_element_type=jnp.float32)
                    + b2_ref[...], 0.0)
    y = x + jnp.dot(h, w3_ref[...], preferred_element_type=jnp.float32) + b3_ref[...]
    mu = jnp.mean(y, axis=-1, keepdims=True)
    var = jnp.mean((y - mu) ** 2, axis=-1, keepdims=True)
    o_ref[...] = (y - mu) / jnp.sqrt(var + 1e-5) * g_ref[...] + be_ref[...]


def node_transition(x, w1, b1, w2, b2, w3, b3, g, be, bm=1024):
    m, k = x.shape
    mp = _ceil_to(m, bm)
    xp = _pad_rows(x, mp)
    row = lambda v: v.reshape(1, -1)
    out = pl.pallas_call(
        _node_trans_kernel,
        grid=(mp // bm,),
        in_specs=[
            pl.BlockSpec((bm, k), lambda i: (i, 0)),
            pl.BlockSpec((C_S, C_S), lambda i: (0, 0)),
            pl.BlockSpec((1, C_S), lambda i: (0, 0)),
            pl.BlockSpec((C_S, C_S), lambda i: (0, 0)),
            pl.BlockSpec((1, C_S), lambda i: (0, 0)),
            pl.BlockSpec((C_S, C_S), lambda i: (0, 0)),
            pl.BlockSpec((1, C_S), lambda i: (0, 0)),
            pl.BlockSpec((1, C_S), lambda i: (0, 0)),
            pl.BlockSpec((1, C_S), lambda i: (0, 0)),
        ],
        out_specs=pl.BlockSpec((bm, C_S), lambda i: (i, 0)),
        out_shape=jax.ShapeDtypeStruct((mp, C_S), jnp.float32),
    )(xp, w1, row(b1), w2, row(b2), w3, row(b3), row(g), row(be))
    return out[:m]


def _edge_trans_kernel(hs_ref, hd_ref, z_ref, w1a_ref, w1b_ref, w1c_ref,
                       b1_ref, w2_ref, b2_ref, g_ref, be_ref, o_ref):
    acc = jnp.dot(hs_ref[...], w1a_ref[...], preferred_element_type=jnp.float32)
    acc += jnp.dot(hd_ref[...], w1b_ref[...], preferred_element_type=jnp.float32)
    acc += jnp.dot(z_ref[...], w1c_ref[...], preferred_element_type=jnp.float32)
    h = jnp.maximum(acc + b1_ref[...], 0.0)
    y = jnp.dot(h, w2_ref[...], preferred_element_type=jnp.float32) + b2_ref[...]
    mu = jnp.mean(y, axis=-1, keepdims=True)
    var = jnp.mean((y - mu) ** 2, axis=-1, keepdims=True)
    o_ref[...] = (y - mu) / jnp.sqrt(var + 1e-5) * g_ref[...] + be_ref[...]


def edge_transition(hs, hd, z, p, bm=2048):
    m = z.shape[0]
    mp = _ceil_to(m, bm)
    w1a = p['w1'][:C_Z]
    w1b = p['w1'][C_Z:2 * C_Z]
    w1c = p['w1'][2 * C_Z:]
    row = lambda v: v.reshape(1, -1)
    out = pl.pallas_call(
        _edge_trans_kernel,
        grid=(mp // bm,),
        in_specs=[
            pl.BlockSpec((bm, C_Z), lambda i: (i, 0)),
            pl.BlockSpec((bm, C_Z), lambda i: (i, 0)),
            pl.BlockSpec((bm, C_Z), lambda i: (i, 0)),
            pl.BlockSpec((C_Z, 2 * C_Z), lambda i: (0, 0)),
            pl.BlockSpec((C_Z, 2 * C_Z), lambda i: (0, 0)),
            pl.BlockSpec((C_Z, 2 * C_Z), lambda i: (0, 0)),
            pl.BlockSpec((1, 2 * C_Z), lambda i: (0, 0)),
            pl.BlockSpec((2 * C_Z, C_Z), lambda i: (0, 0)),
            pl.BlockSpec((1, C_Z), lambda i: (0, 0)),
            pl.BlockSpec((1, C_Z), lambda i: (0, 0)),
            pl.BlockSpec((1, C_Z), lambda i: (0, 0)),
        ],
        out_specs=pl.BlockSpec((bm, C_Z), lambda i: (i, 0)),
        out_shape=jax.ShapeDtypeStruct((mp, C_Z), jnp.float32),
    )(_pad_rows(hs, mp), _pad_rows(hd, mp), _pad_rows(z, mp),
      w1a, w1b, w1c, row(p['b1']), p['w2'], row(p['b2']),
      row(p['lng']), row(p['lnb']))
    return out[:m]


def _edge_logits_kernel(qd_ref, ks_ref, qpd_ref, kps_ref, b_ref, mk_ref,
                        sh_ref, sp_ref, gam_ref, o_ref):
    scal = jnp.dot(qd_ref[...] * ks_ref[...], sh_ref[...],
                   preferred_element_type=jnp.float32) * (1.0 / np.sqrt(C_HID))
    diff = qpd_ref[...] - kps_ref[...]
    d2 = jnp.dot(diff * diff, sp_ref[...], preferred_element_type=jnp.float32)
    wl = np.sqrt(1.0 / 3.0)
    wc = np.sqrt(2.0 / (9.0 * QK))
    logits = wl * (scal + b_ref[...]) - (wl * wc * 0.5) * gam_ref[...] * d2
    o_ref[...] = logits + (mk_ref[...] - 1.0) * 1e5


def edge_logits(qd, ks, qpd, kps, b, mask_src, gamma, bm=2048):
    m = qd.shape[0]
    mp = _ceil_to(m, bm)
    sel_h = jnp.repeat(jnp.eye(H, dtype=jnp.float32), C_HID, axis=0)
    sel_p = jnp.repeat(jnp.eye(H, dtype=jnp.float32), QK * 3, axis=0)
    out = pl.pallas_call(
        _edge_logits_kernel,
        grid=(mp // bm,),
        in_specs=[
            pl.BlockSpec((bm, H * C_HID), lambda i: (i, 0)),
            pl.BlockSpec((bm, H * C_HID), lambda i: (i, 0)),
            pl.BlockSpec((bm, H * QK * 3), lambda i: (i, 0)),
            pl.BlockSpec((bm, H * QK * 3), lambda i: (i, 0)),
            pl.BlockSpec((bm, H), lambda i: (i, 0)),
            pl.BlockSpec((bm, 1), lambda i: (i, 0)),
            pl.BlockSpec((H * C_HID, H), lambda i: (0, 0)),
            pl.BlockSpec((H * QK * 3, H), lambda i: (0, 0)),
            pl.BlockSpec((1, H), lambda i: (0, 0)),
        ],
        out_specs=pl.BlockSpec((bm, H), lambda i: (i, 0)),
        out_shape=jax.ShapeDtypeStruct((mp, H), jnp.float32),
    )(_pad_rows(qd, mp), _pad_rows(ks, mp), _pad_rows(qpd, mp),
      _pad_rows(kps, mp), _pad_rows(b, mp), _pad_rows(mask_src[:, None], mp),
      sel_h, sel_p, gamma.reshape(1, H))
    return out[:m]


def _ln(x, g, b):
    m = x.mean(-1, keepdims=True)
    v = ((x - m) ** 2).mean(-1, keepdims=True)
    return (x - m) / jnp.sqrt(v + 1e-5) * g + b


def _quat_to_rot(q):
    w, x, y, z = q[:, 0], q[:, 1], q[:, 2], q[:, 3]
    R = jnp.stack([
        1 - 2 * (y * y + z * z), 2 * (x * y - w * z), 2 * (x * z + w * y),
        2 * (x * y + w * z), 1 - 2 * (x * x + z * z), 2 * (y * z - w * x),
        2 * (x * z - w * y), 2 * (y * z + w * x), 1 - 2 * (x * x + y * y)], axis=-1)
    return R.reshape(-1, 3, 3)


def _quat_mul(a, b):
    aw, ax, ay, az = a[:, 0], a[:, 1], a[:, 2], a[:, 3]
    bw, bx, by, bz = b[:, 0], b[:, 1], b[:, 2], b[:, 3]
    return jnp.stack([
        aw * bw - ax * bx - ay * by - az * bz,
        aw * bx + ax * bw + ay * bz - az * by,
        aw * by - ax * bz + ay * bw + az * bx,
        aw * bz + ax * by - ay * bx + az * bw], axis=-1)


def _pos_embed(idx, d):
    half = d // 2
    freq = jnp.exp(-np.log(10000.0) * jnp.arange(half) / half)
    ang = idx[:, None].astype(jnp.float32) * freq[None, :]
    return jnp.concatenate([jnp.cos(ang), jnp.sin(ang)], axis=-1)


def _rbf(x, n, lo, hi):
    mu = jnp.linspace(lo, hi, n)
    sig = (hi - lo) / n
    return jnp.exp(-((x[..., None] - mu) ** 2) / (2 * sig ** 2))


def _edge_feats(X, ei, residx):
    src, dst = ei[0], ei[1]
    d = jnp.sqrt(jnp.sum((X[dst] - X[src]) ** 2, axis=-1) + 1e-8)
    rbf = _rbf(d, C_Z // 2, 0.0, 20.0)
    pe = _pos_embed((residx[dst] - residx[src]).astype(jnp.float32), C_Z // 2)
    return jnp.concatenate([rbf, pe], axis=-1)


def _ipa(s, z, ei, q_quat, trans, mask, p):
    n = s.shape[0]
    src, dst = ei[0], ei[1]
    R = _quat_to_rot(q_quat)
    q = pmm(s, p['wq'])
    k = pmm(s, p['wk'])
    v = pmm(s, p['wv'])
    qp = pmm(s, p['wqp']).reshape(n, H, QK, 3)
    kp = pmm(s, p['wkp']).reshape(n, H, QK, 3)
    vp = pmm(s, p['wvp']).reshape(n, H, V, 3)
    qp = jnp.einsum('nij,nhpj->nhpi', R, qp) + trans[:, None, None, :]
    kp = jnp.einsum('nij,nhpj->nhpi', R, kp) + trans[:, None, None, :]
    vp = jnp.einsum('nij,nhpj->nhpi', R, vp) + trans[:, None, None, :]
    b = pmm(z, p['wb'])
    gamma = jax.nn.softplus(p['headw'])

    logits = edge_logits(q[dst], k[src], qp[dst].reshape(-1, H * QK * 3),
                         kp[src].reshape(-1, H * QK * 3), b, mask[src], gamma)
    m = jax.ops.segment_max(logits, dst, num_segments=n)
    m = jnp.where(jnp.isfinite(m), m, 0.0)
    e = jnp.exp(logits - m[dst])
    den = jax.ops.segment_sum(e, dst, num_segments=n)
    a = e / (den[dst] + 1e-9)
    o = jax.ops.segment_sum(a[:, :, None] * v[src].reshape(-1, H, C_HID), dst,
                            num_segments=n)
    opg = jax.ops.segment_sum(a[:, :, None] * vp[src].reshape(-1, H, V * 3), dst,
                              num_segments=n).reshape(n, H, V, 3)
    op = jnp.einsum('nji,nhpj->nhpi', R, opg - trans[:, None, None, :])
    opn = jnp.sqrt(jnp.sum(op ** 2, axis=-1) + 1e-8)
    opair = jax.ops.segment_sum(a[:, :, None] * z[:, None, :], dst, num_segments=n)
    cat = jnp.concatenate([o.reshape(n, -1), op.reshape(n, -1),
                           opn.reshape(n, -1), opair.reshape(n, -1)], axis=-1)
    return pmm(cat, p['wout'], p['bout'])


def _edge_trans(x, z, ei, p):
    src, dst = ei[0], ei[1]
    h = pmm(x, p['wdown'], p['bdown'])
    return edge_transition(h[src], h[dst], z, p)


def _forward(t, x_mask, noising_mask, rigids_7, residx, edge_index,
             seq_edge_index, params):
    q = rigids_7[:, :4]
    q = q / jnp.sqrt(jnp.sum(q ** 2, axis=-1, keepdims=True) + 1e-8)
    tr = rigids_7[:, 4:]
    center = jnp.mean(tr, axis=0, keepdims=True)
    tr = tr - center
    ef = _edge_feats(tr, edge_index, residx)
    sef = _edge_feats(tr, seq_edge_index, residx)
    ft = _rbf(t, HT, 0.0, 1.0)
    et = jax.nn.relu(pmm(ft, params['tm_w1'], params['tm_b1']))
    et = jax.nn.relu(pmm(et, params['tm_w2'], params['tm_b2']))
    rp = _pos_embed(residx.astype(jnp.float32), C_S)
    node_in = jnp.concatenate([rp, et, noising_mask.astype(jnp.float32)[:, None]],
                              axis=-1)
    node = pmm(node_in, params['emb_w'], params['emb_b'])
    valid = (~x_mask).astype(jnp.float32)[:, None]
    node = node * valid
    tr = tr * 0.1
    maskf = (~x_mask).astype(jnp.float32)
    nm = noising_mask.astype(jnp.float32)[:, None]
    for lp in params['layers']:
        u = _ipa(node, ef, edge_index, q, tr, maskf, lp['ipa_sp']) * valid
        node = _ln(node + u, lp['ln1g'], lp['ln1b'])
        u = _ipa(node, sef, seq_edge_index, q, tr, maskf, lp['ipa_sq']) * valid
        node = _ln(node + u, lp['ln2g'], lp['ln2b'])
        node = node_transition(node, lp['nt_w1'], lp['nt_b1'], lp['nt_w2'],
                               lp['nt_b2'], lp['nt_w3'], lp['nt_b3'],
                               lp['ntlng'], lp['ntlnb'])
        node = node * valid
        upd = (pmm(node * nm, lp['bb_w'], lp['bb_b'])) * nm
        uq = jnp.concatenate([jnp.ones((node.shape[0], 1)), upd[:, :3]], axis=-1)
        uq = uq / jnp.sqrt(jnp.sum(uq ** 2, axis=-1, keepdims=True))
        R = _quat_to_rot(q)
        tr = tr + jnp.einsum('nij,nj->ni', R, upd[:, 3:])
        q = _quat_mul(q, uq)
        q = q / jnp.sqrt(jnp.sum(q ** 2, axis=-1, keepdims=True) + 1e-8)
        ef = _edge_trans(node, ef, edge_index, lp['et_sp'])
        sef = _edge_trans(node, sef, seq_edge_index, lp['et_sq'])
    raw = pmm(jax.nn.relu(pmm(node, params['tor_w1'], params['tor_b1'])),
              params['tor_w2'], params['tor_b2'])
    psi = raw / jnp.sqrt(jnp.sum(raw ** 2, axis=-1, keepdims=True) + 1e-8)
    tr = tr * 10.0 + center
    return node, jnp.concatenate([q, tr], axis=-1), psi


def kernel(t, x_mask, noising_mask, rigids_7, residx, edge_index,
           seq_edge_index, params):
    return _forward(t, x_mask, noising_mask, rigids_7, residx, edge_index,
                    seq_edge_index, params)


# speedup vs baseline: 10.8855x; 10.1185x over previous
import functools

import jax
import jax.numpy as jnp
import numpy as np
from jax.experimental import pallas as pl

C_S = 256; C_Z = 64; C_HID = 16; H = 8; QK = 4; V = 8; NL = 2; HT = 64; SH = 128


def _ceil_to(x, m):
    return (x + m - 1) // m * m


def _pad_rows(x, mp):
    m = x.shape[0]
    if m == mp:
        return x
    return jnp.pad(x, ((0, mp - m),) + ((0, 0),) * (x.ndim - 1))


def _mm_kernel(x_ref, w_ref, b_ref, o_ref, *, act):
    acc = jnp.dot(x_ref[...], w_ref[...], preferred_element_type=jnp.float32)
    acc = acc + b_ref[...]
    if act == "relu":
        acc = jnp.maximum(acc, 0.0)
    o_ref[...] = acc


def pmm(x, w, b=None, act="none", bm=1024):
    m, k = x.shape
    n = w.shape[1]
    mp = _ceil_to(m, bm)
    xp = _pad_rows(x, mp)
    if b is None:
        b = jnp.zeros((n,), jnp.float32)
    b2 = b.reshape(1, n)
    out = pl.pallas_call(
        functools.partial(_mm_kernel, act=act),
        grid=(mp // bm,),
        in_specs=[
            pl.BlockSpec((bm, k), lambda i: (i, 0)),
            pl.BlockSpec((k, n), lambda i: (0, 0)),
            pl.BlockSpec((1, n), lambda i: (0, 0)),
        ],
        out_specs=pl.BlockSpec((bm, n), lambda i: (i, 0)),
        out_shape=jax.ShapeDtypeStruct((mp, n), jnp.float32),
    )(xp, w, b2)
    return out[:m]


def _node_trans_kernel(x_ref, w1_ref, b1_ref, w2_ref, b2_ref, w3_ref, b3_ref,
                       g_ref, be_ref, o_ref):
    x = x_ref[...]
    h = jnp.maximum(jnp.dot(x, w1_ref[...], preferred_element_type=jnp.float32)
                    + b1_ref[...], 0.0)
    h = jnp.maximum(jnp.dot(h, w2_ref[...], preferred_element_type=jnp.float32)
                    + b2_ref[...], 0.0)
    y = x + jnp.dot(h, w3_ref[...], preferred_element_type=jnp.float32) + b3_ref[...]
    mu = jnp.mean(y, axis=-1, keepdims=True)
    var = jnp.mean((y - mu) ** 2, axis=-1, keepdims=True)
    o_ref[...] = (y - mu) / jnp.sqrt(var + 1e-5) * g_ref[...] + be_ref[...]


def node_transition(x, w1, b1, w2, b2, w3, b3, g, be, bm=1024):
    m, k = x.shape
    mp = _ceil_to(m, bm)
    xp = _pad_rows(x, mp)
    row = lambda v: v.reshape(1, -1)
    out = pl.pallas_call(
        _node_trans_kernel,
        grid=(mp // bm,),
        in_specs=[
            pl.BlockSpec((bm, k), lambda i: (i, 0)),
            pl.BlockSpec((C_S, C_S), lambda i: (0, 0)),
            pl.BlockSpec((1, C_S), lambda i: (0, 0)),
            pl.BlockSpec((C_S, C_S), lambda i: (0, 0)),
            pl.BlockSpec((1, C_S), lambda i: (0, 0)),
            pl.BlockSpec((C_S, C_S), lambda i: (0, 0)),
            pl.BlockSpec((1, C_S), lambda i: (0, 0)),
            pl.BlockSpec((1, C_S), lambda i: (0, 0)),
            pl.BlockSpec((1, C_S), lambda i: (0, 0)),
        ],
        out_specs=pl.BlockSpec((bm, C_S), lambda i: (i, 0)),
        out_shape=jax.ShapeDtypeStruct((mp, C_S), jnp.float32),
    )(xp, w1, row(b1), w2, row(b2), w3, row(b3), row(g), row(be))
    return out[:m]


def _edge_trans_kernel(hs_ref, hd_ref, z_ref, w1a_ref, w1b_ref, w1c_ref,
                       b1_ref, w2_ref, b2_ref, g_ref, be_ref, o_ref):
    acc = jnp.dot(hs_ref[...], w1a_ref[...], preferred_element_type=jnp.float32)
    acc += jnp.dot(hd_ref[...], w1b_ref[...], preferred_element_type=jnp.float32)
    acc += jnp.dot(z_ref[...], w1c_ref[...], preferred_element_type=jnp.float32)
    h = jnp.maximum(acc + b1_ref[...], 0.0)
    y = jnp.dot(h, w2_ref[...], preferred_element_type=jnp.float32) + b2_ref[...]
    mu = jnp.mean(y, axis=-1, keepdims=True)
    var = jnp.mean((y - mu) ** 2, axis=-1, keepdims=True)
    o_ref[...] = (y - mu) / jnp.sqrt(var + 1e-5) * g_ref[...] + be_ref[...]


def edge_transition(hs, hd, z, p, bm=2048):
    m = z.shape[0]
    mp = _ceil_to(m, bm)
    w1a = p['w1'][:C_Z]
    w1b = p['w1'][C_Z:2 * C_Z]
    w1c = p['w1'][2 * C_Z:]
    row = lambda v: v.reshape(1, -1)
    out = pl.pallas_call(
        _edge_trans_kernel,
        grid=(mp // bm,),
        in_specs=[
            pl.BlockSpec((bm, C_Z), lambda i: (i, 0)),
            pl.BlockSpec((bm, C_Z), lambda i: (i, 0)),
            pl.BlockSpec((bm, C_Z), lambda i: (i, 0)),
            pl.BlockSpec((C_Z, 2 * C_Z), lambda i: (0, 0)),
            pl.BlockSpec((C_Z, 2 * C_Z), lambda i: (0, 0)),
            pl.BlockSpec((C_Z, 2 * C_Z), lambda i: (0, 0)),
            pl.BlockSpec((1, 2 * C_Z), lambda i: (0, 0)),
            pl.BlockSpec((2 * C_Z, C_Z), lambda i: (0, 0)),
            pl.BlockSpec((1, C_Z), lambda i: (0, 0)),
            pl.BlockSpec((1, C_Z), lambda i: (0, 0)),
            pl.BlockSpec((1, C_Z), lambda i: (0, 0)),
        ],
        out_specs=pl.BlockSpec((bm, C_Z), lambda i: (i, 0)),
        out_shape=jax.ShapeDtypeStruct((mp, C_Z), jnp.float32),
    )(_pad_rows(hs, mp), _pad_rows(hd, mp), _pad_rows(z, mp),
      w1a, w1b, w1c, row(p['b1']), p['w2'], row(p['b2']),
      row(p['lng']), row(p['lnb']))
    return out[:m]


def _edge_logits_kernel(qd_ref, ks_ref, qpd_ref, kps_ref, b_ref, mk_ref,
                        sh_ref, sp_ref, gam_ref, o_ref):
    scal = jnp.dot(qd_ref[...] * ks_ref[...], sh_ref[...],
                   preferred_element_type=jnp.float32) * (1.0 / np.sqrt(C_HID))
    diff = qpd_ref[...] - kps_ref[...]
    d2 = jnp.dot(diff * diff, sp_ref[...], preferred_element_type=jnp.float32)
    wl = np.sqrt(1.0 / 3.0)
    wc = np.sqrt(2.0 / (9.0 * QK))
    logits = wl * (scal + b_ref[...]) - (wl * wc * 0.5) * gam_ref[...] * d2
    o_ref[...] = logits + (mk_ref[...] - 1.0) * 1e5


def edge_logits(qd, ks, qpd, kps, b, mask_src, gamma, bm=2048):
    m = qd.shape[0]
    mp = _ceil_to(m, bm)
    sel_h = jnp.repeat(jnp.eye(H, dtype=jnp.float32), C_HID, axis=0)
    sel_p = jnp.repeat(jnp.eye(H, dtype=jnp.float32), QK * 3, axis=0)
    out = pl.pallas_call(
        _edge_logits_kernel,
        grid=(mp // bm,),
        in_specs=[
            pl.BlockSpec((bm, H * C_HID), lambda i: (i, 0)),
            pl.BlockSpec((bm, H * C_HID), lambda i: (i, 0)),
            pl.BlockSpec((bm, H * QK * 3), lambda i: (i, 0)),
            pl.BlockSpec((bm, H * QK * 3), lambda i: (i, 0)),
            pl.BlockSpec((bm, H), lambda i: (i, 0)),
            pl.BlockSpec((bm, 1), lambda i: (i, 0)),
            pl.BlockSpec((H * C_HID, H), lambda i: (0, 0)),
            pl.BlockSpec((H * QK * 3, H), lambda i: (0, 0)),
            pl.BlockSpec((1, H), lambda i: (0, 0)),
        ],
        out_specs=pl.BlockSpec((bm, H), lambda i: (i, 0)),
        out_shape=jax.ShapeDtypeStruct((mp, H), jnp.float32),
    )(_pad_rows(qd, mp), _pad_rows(ks, mp), _pad_rows(qpd, mp),
      _pad_rows(kps, mp), _pad_rows(b, mp), _pad_rows(mask_src[:, None], mp),
      sel_h, sel_p, gamma.reshape(1, H))
    return out[:m]


BE = 512
TN = 256
N_NODES = 10000
E_EDGES = 160000
NBLK = (N_NODES + TN - 1) // TN
NPAD = NBLK * TN
CT = E_EDGES // BE + 1 + NBLK
EP = CT * BE


def build_edge_layout(ei):
    src, dst = ei[0].astype(jnp.int32), ei[1].astype(jnp.int32)
    order = jnp.argsort(dst)
    src_s = jnp.concatenate([src[order], jnp.zeros((1,), jnp.int32)])
    dst_s = jnp.concatenate([dst[order], -jnp.ones((1,), jnp.int32)])
    blk_edges = jnp.searchsorted(dst_s[:E_EDGES],
                                 jnp.arange(NBLK + 1) * TN).astype(jnp.int32)
    cnt = blk_edges[1:] - blk_edges[:-1]
    nch = jnp.maximum(1, (cnt + BE - 1) // BE)
    choff = jnp.concatenate([jnp.zeros((1,), jnp.int32),
                             jnp.cumsum(nch).astype(jnp.int32)])
    c = jnp.arange(CT, dtype=jnp.int32)
    b_of_c = jnp.clip(jnp.searchsorted(choff, c, side='right') - 1, 0, NBLK - 1)
    j = c - choff[b_of_c]
    start = blk_edges[b_of_c] + j * BE
    rows = start[:, None] + jnp.arange(BE, dtype=jnp.int32)[None, :]
    in_range = (rows < blk_edges[b_of_c + 1][:, None]) & (c < choff[-1])[:, None]
    gidx = jnp.where(in_range, rows, E_EDGES).reshape(-1)
    src_pad = src_s[gidx]
    dst_pad = dst_s[gidx]
    dstloc = dst_pad - (b_of_c[:, None] * TN).repeat(BE, axis=1).reshape(-1)
    firstflag = ((j == 0) & (c < choff[-1])).astype(jnp.int32)
    return {
        'src': src_pad, 'dst': dst_pad,
        'dstloc_col': dstloc.reshape(EP, 1),
        'dstloc_row': dstloc.reshape(CT, 1, BE),
        'bmap': b_of_c.astype(jnp.int32),
        'ff': firstflag,
    }


def _ipa_edge_kernel(bmap_ref, ff_ref,
                     dlc_ref, dlr_ref, ks_ref, kps_ref, vs_ref, vps_ref,
                     z_ref, b_ref, q_ref, qp_ref,
                     selh_ref, selp_ref, reph_ref, repp_ref, repz_ref, gam_ref,
                     den_ref, ov_ref, ovp_ref, opair_ref):
    i = pl.program_id(0)
    dl = dlc_ref[...]
    iota_l = jax.lax.broadcasted_iota(jnp.int32, (BE, TN), 1)
    D = (dl == iota_l).astype(jnp.float32)
    dlr = dlr_ref[...].reshape(1, BE)
    iota_t = jax.lax.broadcasted_iota(jnp.int32, (TN, BE), 0)
    Dt = (dlr == iota_t).astype(jnp.float32)

    qd = jnp.dot(D, q_ref[...], preferred_element_type=jnp.float32)
    qpd = jnp.dot(D, qp_ref[...], preferred_element_type=jnp.float32)
    scal = jnp.dot(qd * ks_ref[...], selh_ref[...],
                   preferred_element_type=jnp.float32) * (1.0 / np.sqrt(C_HID))
    diff = qpd - kps_ref[...]
    d2 = jnp.dot(diff * diff, selp_ref[...], preferred_element_type=jnp.float32)
    wl = np.sqrt(1.0 / 3.0)
    logits = wl * (scal + b_ref[...]) - gam_ref[...] * d2
    e = jnp.exp(logits)

    erep_h = jnp.dot(e, reph_ref[...], preferred_element_type=jnp.float32)
    erep_p = jnp.dot(e, repp_ref[...], preferred_element_type=jnp.float32)
    erep_z = jnp.dot(e, repz_ref[...], preferred_element_type=jnp.float32)
    z = z_ref[...]
    ztile = jnp.concatenate([z] * H, axis=1)

    den_c = jnp.dot(Dt, e, preferred_element_type=jnp.float32)
    ov_c = jnp.dot(Dt, erep_h * vs_ref[...], preferred_element_type=jnp.float32)
    ovp_c = jnp.dot(Dt, erep_p * vps_ref[...], preferred_element_type=jnp.float32)
    opair_c = jnp.dot(Dt, erep_z * ztile, preferred_element_type=jnp.float32)

    @pl.when(ff_ref[i] == 1)
    def _init():
        den_ref[...] = den_c
        ov_ref[...] = ov_c
        ovp_ref[...] = ovp_c
        opair_ref[...] = opair_c

    @pl.when(ff_ref[i] == 0)
    def _acc():
        den_ref[...] += den_c
        ov_ref[...] += ov_c
        ovp_ref[...] += ovp_c
        opair_ref[...] += opair_c


def ipa_edge_phase(layout, ks, kps, vs, vps, z, b, q_nodes, qp_nodes, gam):
    from jax.experimental.pallas import tpu as pltpu
    sel_h = jnp.repeat(jnp.eye(H, dtype=jnp.float32), C_HID, axis=0)
    sel_p = jnp.repeat(jnp.eye(H, dtype=jnp.float32), QK * 3, axis=0)
    rep_h = jnp.repeat(jnp.eye(H, dtype=jnp.float32), C_HID, axis=1)
    rep_p = jnp.repeat(jnp.eye(H, dtype=jnp.float32), V * 3, axis=1)
    rep_z = jnp.repeat(jnp.eye(H, dtype=jnp.float32), C_Z, axis=1)
    ed = lambda f: pl.BlockSpec((BE, f), lambda i, bmap, ff: (i, 0))
    nd = lambda f: pl.BlockSpec((TN, f), lambda i, bmap, ff: (bmap[i], 0))
    cs = lambda r, f: pl.BlockSpec((r, f), lambda i, bmap, ff: (0, 0))
    grid_spec = pltpu.PrefetchScalarGridSpec(
        num_scalar_prefetch=2,
        grid=(CT,),
        in_specs=[
            ed(1),
            pl.BlockSpec((1, 1, BE), lambda i, bmap, ff: (i, 0, 0)),
            ed(H * C_HID), ed(H * QK * 3), ed(H * C_HID), ed(H * V * 3),
            ed(C_Z), ed(H),
            nd(H * C_HID), nd(H * QK * 3),
            cs(H * C_HID, H), cs(H * QK * 3, H),
            cs(H, H * C_HID), cs(H, H * V * 3), cs(H, H * C_Z), cs(1, H),
        ],
        out_specs=[nd(H), nd(H * C_HID), nd(H * V * 3), nd(H * C_Z)],
    )
    outs = pl.pallas_call(
        _ipa_edge_kernel,
        grid_spec=grid_spec,
        out_shape=[
            jax.ShapeDtypeStruct((NPAD, H), jnp.float32),
            jax.ShapeDtypeStruct((NPAD, H * C_HID), jnp.float32),
            jax.ShapeDtypeStruct((NPAD, H * V * 3), jnp.float32),
            jax.ShapeDtypeStruct((NPAD, H * C_Z), jnp.float32),
        ],
    )(layout['bmap'], layout['ff'],
      layout['dstloc_col'], layout['dstloc_row'],
      ks, kps, vs, vps, z, b,
      _pad_rows(q_nodes, NPAD), _pad_rows(qp_nodes, NPAD),
      sel_h, sel_p, rep_h, rep_p, rep_z, gam.reshape(1, H))
    return outs


def _ln(x, g, b):
    m = x.mean(-1, keepdims=True)
    v = ((x - m) ** 2).mean(-1, keepdims=True)
    return (x - m) / jnp.sqrt(v + 1e-5) * g + b


def _quat_to_rot(q):
    w, x, y, z = q[:, 0], q[:, 1], q[:, 2], q[:, 3]
    R = jnp.stack([
        1 - 2 * (y * y + z * z), 2 * (x * y - w * z), 2 * (x * z + w * y),
        2 * (x * y + w * z), 1 - 2 * (x * x + z * z), 2 * (y * z - w * x),
        2 * (x * z - w * y), 2 * (y * z + w * x), 1 - 2 * (x * x + y * y)], axis=-1)
    return R.reshape(-1, 3, 3)


def _quat_mul(a, b):
    aw, ax, ay, az = a[:, 0], a[:, 1], a[:, 2], a[:, 3]
    bw, bx, by, bz = b[:, 0], b[:, 1], b[:, 2], b[:, 3]
    return jnp.stack([
        aw * bw - ax * bx - ay * by - az * bz,
        aw * bx + ax * bw + ay * bz - az * by,
        aw * by - ax * bz + ay * bw + az * bx,
        aw * bz + ax * by - ay * bx + az * bw], axis=-1)


def _pos_embed(idx, d):
    half = d // 2
    freq = jnp.exp(-np.log(10000.0) * jnp.arange(half) / half)
    ang = idx[:, None].astype(jnp.float32) * freq[None, :]
    return jnp.concatenate([jnp.cos(ang), jnp.sin(ang)], axis=-1)


def _rbf(x, n, lo, hi):
    mu = jnp.linspace(lo, hi, n)
    sig = (hi - lo) / n
    return jnp.exp(-((x[..., None] - mu) ** 2) / (2 * sig ** 2))


def _edge_feats(X, src, dst, residx):
    d = jnp.sqrt(jnp.sum((X[dst] - X[src]) ** 2, axis=-1) + 1e-8)
    rbf = _rbf(d, C_Z // 2, 0.0, 20.0)
    pe = _pos_embed((residx[dst] - residx[src]).astype(jnp.float32), C_Z // 2)
    return jnp.concatenate([rbf, pe], axis=-1)


def _ipa(s, z, layout, q_quat, trans, p):
    n = s.shape[0]
    src = layout['src']
    R = _quat_to_rot(q_quat)
    wcat = jnp.concatenate([p['wq'], p['wk'], p['wv'], p['wqp'], p['wkp'],
                            p['wvp']], axis=1)
    proj = pmm(s, wcat)
    q, k, v = proj[:, :128], proj[:, 128:256], proj[:, 256:384]
    qp = proj[:, 384:480].reshape(n, H, QK, 3)
    kp = proj[:, 480:576].reshape(n, H, QK, 3)
    vp = proj[:, 576:768].reshape(n, H, V, 3)
    qp = jnp.einsum('nij,nhpj->nhpi', R, qp) + trans[:, None, None, :]
    kp = jnp.einsum('nij,nhpj->nhpi', R, kp) + trans[:, None, None, :]
    vp = jnp.einsum('nij,nhpj->nhpi', R, vp) + trans[:, None, None, :]
    b = pmm(z, p['wb'])
    gamma = jax.nn.softplus(p['headw'])
    gam_scaled = gamma * (np.sqrt(1.0 / 3.0) * np.sqrt(2.0 / (9.0 * QK)) * 0.5)

    ks = k[src]
    kps = kp.reshape(n, -1)[src]
    vs = v[src]
    vps = vp.reshape(n, -1)[src]
    den, ov, ovp, opair = ipa_edge_phase(
        layout, ks, kps, vs, vps, z, b, q, qp.reshape(n, -1), gam_scaled)
    den = den[:n]
    inv = 1.0 / (den + 1e-9)
    o = ov[:n] * jnp.repeat(inv, C_HID, axis=1)
    opg = (ovp[:n] * jnp.repeat(inv, V * 3, axis=1)).reshape(n, H, V, 3)
    opair_n = opair[:n] * jnp.repeat(inv, C_Z, axis=1)
    op = jnp.einsum('nji,nhpj->nhpi', R, opg - trans[:, None, None, :])
    opn = jnp.sqrt(jnp.sum(op ** 2, axis=-1) + 1e-8)
    cat = jnp.concatenate([o, op.reshape(n, -1), opn.reshape(n, -1),
                           opair_n], axis=-1)
    return pmm(cat, p['wout'], p['bout'])


def _edge_trans(x, z, layout, p):
    h = pmm(x, p['wdown'], p['bdown'])
    return edge_transition(h[layout['src']], h[layout['dst']], z, p)


def _forward(t, x_mask, noising_mask, rigids_7, residx, edge_index,
             seq_edge_index, params):
    q = rigids_7[:, :4]
    q = q / jnp.sqrt(jnp.sum(q ** 2, axis=-1, keepdims=True) + 1e-8)
    tr = rigids_7[:, 4:]
    center = jnp.mean(tr, axis=0, keepdims=True)
    tr = tr - center
    lay_e = build_edge_layout(edge_index)
    lay_s = build_edge_layout(seq_edge_index)
    ef = _edge_feats(tr, lay_e['src'], lay_e['dst'], residx)
    sef = _edge_feats(tr, lay_s['src'], lay_s['dst'], residx)
    ft = _rbf(t, HT, 0.0, 1.0)
    et = jax.nn.relu(pmm(ft, params['tm_w1'], params['tm_b1']))
    et = jax.nn.relu(pmm(et, params['tm_w2'], params['tm_b2']))
    rp = _pos_embed(residx.astype(jnp.float32), C_S)
    node_in = jnp.concatenate([rp, et, noising_mask.astype(jnp.float32)[:, None]],
                              axis=-1)
    node = pmm(node_in, params['emb_w'], params['emb_b'])
    valid = (~x_mask).astype(jnp.float32)[:, None]
    node = node * valid
    tr = tr * 0.1
    maskf = (~x_mask).astype(jnp.float32)
    nm = noising_mask.astype(jnp.float32)[:, None]
    for lp in params['layers']:
        u = _ipa(node, ef, lay_e, q, tr, lp['ipa_sp']) * valid
        node = _ln(node + u, lp['ln1g'], lp['ln1b'])
        u = _ipa(node, sef, lay_s, q, tr, lp['ipa_sq']) * valid
        node = _ln(node + u, lp['ln2g'], lp['ln2b'])
        node = node_transition(node, lp['nt_w1'], lp['nt_b1'], lp['nt_w2'],
                               lp['nt_b2'], lp['nt_w3'], lp['nt_b3'],
                               lp['ntlng'], lp['ntlnb'])
        node = node * valid
        upd = (pmm(node * nm, lp['bb_w'], lp['bb_b'])) * nm
        uq = jnp.concatenate([jnp.ones((node.shape[0], 1)), upd[:, :3]], axis=-1)
        uq = uq / jnp.sqrt(jnp.sum(uq ** 2, axis=-1, keepdims=True))
        R = _quat_to_rot(q)
        tr = tr + jnp.einsum('nij,nj->ni', R, upd[:, 3:])
        q = _quat_mul(q, uq)
        q = q / jnp.sqrt(jnp.sum(q ** 2, axis=-1, keepdims=True) + 1e-8)
        ef = _edge_trans(node, ef, lay_e, lp['et_sp'])
        sef = _edge_trans(node, sef, lay_s, lp['et_sq'])
    raw = pmm(jax.nn.relu(pmm(node, params['tor_w1'], params['tor_b1'])),
              params['tor_w2'], params['tor_b2'])
    psi = raw / jnp.sqrt(jnp.sum(raw ** 2, axis=-1, keepdims=True) + 1e-8)
    tr = tr * 10.0 + center
    return node, jnp.concatenate([q, tr], axis=-1), psi


def kernel(t, x_mask, noising_mask, rigids_7, residx, edge_index,
           seq_edge_index, params):
    return _forward(t, x_mask, noising_mask, rigids_7, residx, edge_index,
                    seq_edge_index, params)
